# baseline (speedup 1.0000x reference)
import functools

import jax
import jax.numpy as jnp
import numpy as np
from jax import lax
from jax.experimental import pallas as pl
from jax.experimental.pallas import tpu as pltpu
from jax.experimental.pallas import tpu_sc as plsc

NN = 10000
EE = 320000
DIN = 128
BN_EPS = 1e-5
NC, NS = 2, 16
NT = NC * NS
PR = 320
NP = NT * PR
CAP = 12288
CH = 48
ECH = 4000
BR = 400
NBLK = NN // BR
NEG = -3.0e38

_MESH = plsc.VectorSubcoreMesh(
    core_axis_name="c", subcore_axis_name="s", num_cores=NC, num_subcores=NS)


def _wid():
    return lax.axis_index("s") * NC + lax.axis_index("c")


def _partition_body(src_hbm, dst_hbm, osrc_hbm, oldst_hbm, nch_hbm,
                    srcb, dstb, osrc_v, oldst_v, miscv):
    w = _wid()
    lo16 = jnp.full((16,), w * PR, jnp.int32)
    iota = lax.iota(jnp.int32, 16)
    zero16 = jnp.zeros((16,), jnp.int32)

    def chunk_body(ci, off):
        pltpu.sync_copy(src_hbm.at[pl.ds(ci * ECH, ECH)], srcb)
        pltpu.sync_copy(dst_hbm.at[pl.ds(ci * ECH, ECH)], dstb)

        def grp(g, off):
            d16 = dstb[pl.ds(g * 16, 16)]
            s16 = srcb[pl.ds(g * 16, 16)]
            ld = d16 - lo16
            m = (ld >= 0) & (ld < PR)
            mi = m.astype(jnp.int32)
            cs = plsc.cumsum(mi)
            pos = off + cs - mi
            ok = m & (pos < CAP - CH)
            plsc.store_scatter(osrc_v, [pos], s16, mask=ok)
            plsc.store_scatter(oldst_v, [pos], ld, mask=ok)
            return off + plsc.all_reduce_population_count(m)

        return lax.fori_loop(0, ECH // 16, grp, off)

    off = lax.fori_loop(0, EE // ECH, chunk_body, zero16)
    pc = ((off + (CH - 1)) // CH) * CH
    for k in range(CH // 16):
        pos = off + k * 16 + iota
        pm = pos < pc
        plsc.store_scatter(osrc_v, [pos], zero16, mask=pm)
        plsc.store_scatter(oldst_v, [pos], zero16 + PR, mask=pm)
    miscv[...] = pc // CH
    pltpu.sync_copy(osrc_v, osrc_hbm.at[pl.ds(w * CAP, CAP)])
    pltpu.sync_copy(oldst_v, oldst_hbm.at[pl.ds(w * CAP, CAP)])
    pltpu.sync_copy(miscv, nch_hbm.at[pl.ds(w * 16, 16)])


_partition = pl.kernel(
    _partition_body,
    out_type=(
        jax.ShapeDtypeStruct((NT * CAP,), jnp.int32),
        jax.ShapeDtypeStruct((NT * CAP,), jnp.int32),
        jax.ShapeDtypeStruct((NT * 16,), jnp.int32),
    ),
    mesh=_MESH,
    compiler_params=pltpu.CompilerParams(needs_layout_passes=False),
    scratch_types=[
        pltpu.VMEM((ECH,), jnp.int32),
        pltpu.VMEM((ECH,), jnp.int32),
        pltpu.VMEM((CAP,), jnp.int32),
        pltpu.VMEM((CAP,), jnp.int32),
        pltpu.VMEM((16,), jnp.int32),
    ],
)


def _vbcast(v, j):
    idx = jnp.full((16, 1), j, jnp.int32)
    dn = lax.GatherDimensionNumbers(
        offset_dims=(), collapsed_slice_dims=(0,), start_index_map=(0,))
    return lax.gather(v, idx, dn, (1,),
                      mode=lax.GatherScatterMode.PROMISE_IN_BOUNDS)


def _make_seg(fanout, half, with_cnt):
    def body(h_hbm, osrc_hbm, oldst_hbm, nch_hbm, *rest):
        if with_cnt:
            sum_hbm, max_hbm, cnt_hbm = rest[:3]
            rest = rest[3:]
        else:
            sum_hbm, max_hbm = rest[:2]
            rest = rest[2:]
        srcb, ldstb, idxb, rows_v = rest[:4]
        acc_s = rest[4]
        acc_m = rest[5]
        acc_c, nchv, sem = rest[6:]
        w = _wid()
        iota = lax.iota(jnp.int32, 16)
        zero16f = jnp.zeros((16,), jnp.float32)
        one0 = (iota < 1).astype(jnp.float32)

        def init(i, _):
            for k in range(8):
                acc_s[i, pl.ds(k * 16, 16)] = zero16f
                acc_m[i, pl.ds(k * 16, 16)] = zero16f + NEG
            acc_c[i, pl.ds(0, 16)] = zero16f
            return 0

        lax.fori_loop(0, PR, init, 0)
        pltpu.sync_copy(nch_hbm.at[pl.ds(w * 16, 16)], nchv)
        nch = jnp.max(nchv[...])

        def chunk(ci, _):
            pltpu.sync_copy(osrc_hbm.at[pl.ds(w * CAP + ci * CH, CH)], srcb)
            pltpu.sync_copy(oldst_hbm.at[pl.ds(w * CAP + ci * CH, CH)], ldstb)
            if fanout == 1:
                gidx = srcb
            else:
                for k in range(CH // 16):
                    v = srcb[pl.ds(k * 16, 16)]
                    idxb[pl.ds(k * 16, 16)] = v * fanout + half
                gidx = idxb
            pltpu.async_copy(h_hbm.at[gidx], rows_v, sem).wait()

            def grp(g, _):
                ld16 = ldstb[pl.ds(g * 16, 16)]
                for j in range(16):
                    e = g * 16 + j
                    ldb = _vbcast(ld16, j)
                    ok = ldb < PR
                    cols = [iota + k * 16 for k in range(8)]
                    mg = [plsc.load_gather(acc_m, [ldb, cols[k]], mask=ok)
                          for k in range(8)]
                    rr = [rows_v[e, pl.ds(k * 16, 16)] for k in range(8)]
                    for k in range(8):
                        plsc.store_scatter(acc_m, [ldb, cols[k]],
                                           jnp.maximum(mg[k], rr[k]), mask=ok)
                    for k in range(8):
                        plsc.addupdate_scatter(acc_s, [ldb, cols[k]], rr[k], mask=ok)
                    if with_cnt:
                        plsc.addupdate_scatter(acc_c, [ldb, iota], one0, mask=ok)
                return 0

            lax.fori_loop(0, CH // 16, grp, 0)
            return 0

        lax.fori_loop(0, nch, chunk, 0)
        pltpu.sync_copy(acc_s, sum_hbm.at[pl.ds(w * PR, PR)])
        pltpu.sync_copy(acc_m, max_hbm.at[pl.ds(w * PR, PR)])
        if with_cnt:
            pltpu.sync_copy(acc_c, cnt_hbm.at[pl.ds(w * PR, PR)])

    outs = [jax.ShapeDtypeStruct((NP, 128), jnp.float32),
            jax.ShapeDtypeStruct((NP, 128), jnp.float32)]
    if with_cnt:
        outs.append(jax.ShapeDtypeStruct((NP, 16), jnp.float32))
    return pl.kernel(
        body,
        out_type=tuple(outs),
        mesh=_MESH,
        compiler_params=pltpu.CompilerParams(needs_layout_passes=False),
        scratch_types=[
            pltpu.VMEM((CH,), jnp.int32),
            pltpu.VMEM((CH,), jnp.int32),
            pltpu.VMEM((CH,), jnp.int32),
            pltpu.VMEM((CH, 128), jnp.float32),
            pltpu.VMEM((PR, 128), jnp.float32),
            pltpu.VMEM((PR, 128), jnp.float32),
            pltpu.VMEM((PR, 16), jnp.float32),
            pltpu.VMEM((16,), jnp.int32),
            pltpu.SemaphoreType.DMA,
        ],
    )


_seg0 = _make_seg(1, 0, True)
_seg1a = _make_seg(2, 0, False)
_seg1b = _make_seg(2, 1, False)


_BNS = float(1.0 / np.sqrt(1.0 + BN_EPS))


def _sage_path(h, agg, wlT, bl, wrT, g, b):
    o = jnp.dot(agg, wlT, preferred_element_type=jnp.float32) + bl
    o = o + jnp.dot(h, wrT, preferred_element_type=jnp.float32)
    n = jnp.sqrt(jnp.sum(o * o, axis=1, keepdims=True))
    o = o / jnp.maximum(n, 1e-12)
    o = g * o * _BNS + b
    return jnp.maximum(o, 0.0)


def _tc0_body(x_ref, s_ref, m_ref, c_ref,
              wlmT, blm, wrmT, gm, bm, wlxT, blx, wrxT, gx, bx, aw,
              h_out):
    h = x_ref[...]
    c0 = c_ref[:, 0:1]
    mean = s_ref[...] / jnp.maximum(c0, 1.0)
    mx = jnp.where(c0 > 0.0, m_ref[...], 0.0)
    hm = _sage_path(h, mean, wlmT[...], blm[...], wrmT[...], gm[...], bm[...])
    hx = _sage_path(h, mx, wlxT[...], blx[...], wrxT[...], gx[...], bx[...])
    a = aw[...]
    e = jnp.exp(a - jnp.max(a))
    wgt = e / jnp.sum(e)
    h_out[...] = wgt[0, 0] * hm + wgt[0, 1] * hx


def _tc1_body(h_ref, sa_ref, sb_ref, ma_ref, mb_ref, c_ref,
              wlmT, blm, wrmT, gm, bm, wlxT, blx, wrxT, gx, bx, aw,
              cw0T, cb0, cg0, cbe0, cw1T, cb1, cg1, cbe1, cw2T, cb2,
              emb_out, log_out, acc):
    i = pl.program_id(0)

    @pl.when(i == 0)
    def _():
        acc[...] = jnp.zeros_like(acc)

    h = h_ref[...]
    c0 = c_ref[:, 0:1]
    s = jnp.concatenate([sa_ref[...], sb_ref[...]], axis=1)
    mx = jnp.concatenate([ma_ref[...], mb_ref[...]], axis=1)
    mean = s / jnp.maximum(c0, 1.0)
    mx = jnp.where(c0 > 0.0, mx, 0.0)
    hm = _sage_path(h, mean, wlmT[...], blm[...], wrmT[...], gm[...], bm[...])
    hx = _sage_path(h, mx, wlxT[...], blx[...], wrxT[...], gx[...], bx[...])
    a = aw[...]
    e = jnp.exp(a - jnp.max(a))
    wgt = e / jnp.sum(e)
    h2 = wgt[0, 0] * hm + wgt[0, 1] * hx
    acc[0:1, :] = acc[0:1, :] + jnp.sum(h2, axis=0, keepdims=True)

    @pl.when(i == NBLK - 1)
    def _():
        emb = acc[0:1, :] * (1.0 / NN)
        z = jnp.dot(emb, cw0T[...], preferred_element_type=jnp.float32) + cb0[...]
        z = jnp.maximum(cg0[...] * z * _BNS + cbe0[...], 0.0)
        z = jnp.dot(z, cw1T[...], preferred_element_type=jnp.float32) + cb1[...]
        z = jnp.maximum(cg1[...] * z * _BNS + cbe1[...], 0.0)
        log_out[...] = jnp.dot(z, cw2T[...], preferred_element_type=jnp.float32) + cb2[...]
        emb_out[...] = emb


def _full(shape):
    return pl.BlockSpec(shape, lambda i: (0, 0))


def _rows(shape):
    return pl.BlockSpec(shape, lambda i: (i, 0))


_tc0 = pl.pallas_call(
    _tc0_body,
    grid=(NBLK,),
    in_specs=[
        _rows((BR, 128)), _rows((BR, 128)), _rows((BR, 128)), _rows((BR, 16)),
        _full((128, 256)), _full((1, 256)), _full((128, 256)),
        _full((1, 256)), _full((1, 256)),
        _full((128, 256)), _full((1, 256)), _full((128, 256)),
        _full((1, 256)), _full((1, 256)),
        _full((1, 2)),
    ],
    out_specs=_rows((BR, 256)),
    out_shape=jax.ShapeDtypeStruct((NN, 256), jnp.float32),
)

_tc1 = pl.pallas_call(
    _tc1_body,
    grid=(NBLK,),
    in_specs=[
        _rows((BR, 256)), _rows((BR, 128)), _rows((BR, 128)),
        _rows((BR, 128)), _rows((BR, 128)), _rows((BR, 16)),
        _full((256, 128)), _full((1, 128)), _full((256, 128)),
        _full((1, 128)), _full((1, 128)),
        _full((256, 128)), _full((1, 128)), _full((256, 128)),
        _full((1, 128)), _full((1, 128)),
        _full((1, 2)),
        _full((128, 256)), _full((1, 256)), _full((1, 256)), _full((1, 256)),
        _full((256, 128)), _full((1, 128)), _full((1, 128)), _full((1, 128)),
        _full((128, 2)), _full((1, 2)),
    ],
    out_specs=[_full((1, 128)), _full((1, 2))],
    out_shape=[jax.ShapeDtypeStruct((1, 128), jnp.float32),
               jax.ShapeDtypeStruct((1, 2), jnp.float32)],
    scratch_shapes=[pltpu.VMEM((8, 128), jnp.float32)],
)


def _row(v):
    return v.reshape(1, -1)


def kernel(x, params, edge_index):
    src = edge_index[0].astype(jnp.int32)
    dst = edge_index[1].astype(jnp.int32)
    osrc, oldst, nch = _partition(src, dst)
    s0, m0, c0 = _seg0(x, osrc, oldst, nch)

    p = params
    h1 = _tc0(
        x, s0[:NN], m0[:NN], c0[:NN],
        p["mean0_Wl"].T, _row(p["mean0_bl"]), p["mean0_Wr"].T,
        _row(p["mean0_gamma"]), _row(p["mean0_beta"]),
        p["max0_Wl"].T, _row(p["max0_bl"]), p["max0_Wr"].T,
        _row(p["max0_gamma"]), _row(p["max0_beta"]),
        _row(p["aggr_w0"]),
    )

    h1v = h1.reshape(NN * 2, 128)
    s1a, m1a = _seg1a(h1v, osrc, oldst, nch)
    s1b, m1b = _seg1b(h1v, osrc, oldst, nch)

    emb, logits = _tc1(
        h1, s1a[:NN], s1b[:NN], m1a[:NN], m1b[:NN], c0[:NN],
        p["mean1_Wl"].T, _row(p["mean1_bl"]), p["mean1_Wr"].T,
        _row(p["mean1_gamma"]), _row(p["mean1_beta"]),
        p["max1_Wl"].T, _row(p["max1_bl"]), p["max1_Wr"].T,
        _row(p["max1_gamma"]), _row(p["max1_beta"]),
        _row(p["aggr_w1"]),
        p["cls_W0"].T, _row(p["cls_b0"]), _row(p["cls_gamma0"]), _row(p["cls_beta0"]),
        p["cls_W1"].T, _row(p["cls_b1"]), _row(p["cls_gamma1"]), _row(p["cls_beta1"]),
        p["cls_W2"].T, _row(p["cls_b2"]),
    )
    return (logits, emb)


# speedup vs baseline: 1.1813x; 1.1813x over previous
import functools

import jax
import jax.numpy as jnp
import numpy as np
from jax import lax
from jax.experimental import pallas as pl
from jax.experimental.pallas import tpu as pltpu
from jax.experimental.pallas import tpu_sc as plsc

NN = 10000
EE = 320000
DIN = 128
BN_EPS = 1e-5
NC, NS = 2, 16
NT = NC * NS
PR = 320
NP = NT * PR
CAP = 12288
CH = 48
ECH = 4000
BR = 400
NBLK = NN // BR
NEG = -3.0e38

_MESH = plsc.VectorSubcoreMesh(
    core_axis_name="c", subcore_axis_name="s", num_cores=NC, num_subcores=NS)


def _wid():
    return lax.axis_index("s") * NC + lax.axis_index("c")


def _partition_body(src_hbm, dst_hbm, osrc_hbm, oldst_hbm, nch_hbm,
                    srcb, dstb, osrc_v, oldst_v, miscv):
    w = _wid()
    lo16 = jnp.full((16,), w * PR, jnp.int32)
    iota = lax.iota(jnp.int32, 16)
    zero16 = jnp.zeros((16,), jnp.int32)

    def chunk_body(ci, off):
        pltpu.sync_copy(src_hbm.at[pl.ds(ci * ECH, ECH)], srcb)
        pltpu.sync_copy(dst_hbm.at[pl.ds(ci * ECH, ECH)], dstb)

        def grp(g, off):
            d16 = dstb[pl.ds(g * 16, 16)]
            s16 = srcb[pl.ds(g * 16, 16)]
            ld = d16 - lo16
            m = (ld >= 0) & (ld < PR)
            mi = m.astype(jnp.int32)
            cs = plsc.cumsum(mi)
            pos = off + cs - mi
            ok = m & (pos < CAP - CH)
            plsc.store_scatter(osrc_v, [pos], s16, mask=ok)
            plsc.store_scatter(oldst_v, [pos], ld, mask=ok)
            return off + plsc.all_reduce_population_count(m)

        return lax.fori_loop(0, ECH // 16, grp, off)

    off = lax.fori_loop(0, EE // ECH, chunk_body, zero16)
    pc = ((off + (CH - 1)) // CH) * CH
    for k in range(CH // 16):
        pos = off + k * 16 + iota
        pm = pos < pc
        plsc.store_scatter(osrc_v, [pos], zero16, mask=pm)
        plsc.store_scatter(oldst_v, [pos], zero16 + PR, mask=pm)
    miscv[...] = pc // CH
    pltpu.sync_copy(osrc_v, osrc_hbm.at[pl.ds(w * CAP, CAP)])
    pltpu.sync_copy(oldst_v, oldst_hbm.at[pl.ds(w * CAP, CAP)])
    pltpu.sync_copy(miscv, nch_hbm.at[pl.ds(w * 16, 16)])


_partition = pl.kernel(
    _partition_body,
    out_type=(
        jax.ShapeDtypeStruct((NT * CAP,), jnp.int32),
        jax.ShapeDtypeStruct((NT * CAP,), jnp.int32),
        jax.ShapeDtypeStruct((NT * 16,), jnp.int32),
    ),
    mesh=_MESH,
    compiler_params=pltpu.CompilerParams(needs_layout_passes=False),
    scratch_types=[
        pltpu.VMEM((ECH,), jnp.int32),
        pltpu.VMEM((ECH,), jnp.int32),
        pltpu.VMEM((CAP,), jnp.int32),
        pltpu.VMEM((CAP,), jnp.int32),
        pltpu.VMEM((16,), jnp.int32),
    ],
)


def _vbcast(v, j):
    idx = jnp.full((16, 1), j, jnp.int32)
    dn = lax.GatherDimensionNumbers(
        offset_dims=(), collapsed_slice_dims=(0,), start_index_map=(0,))
    return lax.gather(v, idx, dn, (1,),
                      mode=lax.GatherScatterMode.PROMISE_IN_BOUNDS)


def _make_seg(fanout, half, with_cnt):
    def _process(acc_s, acc_m, acc_c, ldstb, rows_v, iota, one0):
        def grp(g, _):
            ld16 = ldstb[pl.ds(g * 16, 16)]
            for j in range(16):
                e = g * 16 + j
                ldb = _vbcast(ld16, j)
                ok = ldb < PR
                cols = [iota + k * 16 for k in range(8)]
                mg = [plsc.load_gather(acc_m, [ldb, cols[k]], mask=ok)
                      for k in range(8)]
                rr = [rows_v[e, pl.ds(k * 16, 16)] for k in range(8)]
                for k in range(8):
                    plsc.store_scatter(acc_m, [ldb, cols[k]],
                                       jnp.maximum(mg[k], rr[k]), mask=ok)
                for k in range(8):
                    plsc.addupdate_scatter(acc_s, [ldb, cols[k]], rr[k], mask=ok)
                if acc_c is not None:
                    plsc.addupdate_scatter(acc_c, [ldb, iota], one0, mask=ok)
            return 0

        lax.fori_loop(0, CH // 16, grp, 0)

    def body(h_hbm, osrc_hbm, oldst_hbm, nch_hbm, *rest):
        if with_cnt:
            sum_hbm, max_hbm, cnt_hbm = rest[:3]
            rest = rest[3:]
        else:
            sum_hbm, max_hbm = rest[:2]
            rest = rest[2:]
        w = _wid()
        iota = lax.iota(jnp.int32, 16)
        zero16f = jnp.zeros((16,), jnp.float32)
        one0 = (iota < 1).astype(jnp.float32)

        if with_cnt:
            srcb, ldstb, idxb, rows_v = rest[:4]
            acc_s = rest[4]
            acc_m = rest[5]
            acc_c, nchv, sem = rest[6:]
        else:
            srcb0, srcb1, ldstb0, ldstb1, rows0, rows1 = rest[:6]
            acc_s, acc_m, nchv, sem0, sem1 = rest[6:]
            acc_c = None

        def init(i, _):
            for k in range(8):
                acc_s[i, pl.ds(k * 16, 16)] = zero16f
                acc_m[i, pl.ds(k * 16, 16)] = zero16f + NEG
            if acc_c is not None:
                acc_c[i, pl.ds(0, 16)] = zero16f
            return 0

        lax.fori_loop(0, PR, init, 0)
        pltpu.sync_copy(nch_hbm.at[pl.ds(w * 16, 16)], nchv)
        nch = jnp.max(nchv[...])

        if with_cnt:
            def chunk(ci, _):
                pltpu.sync_copy(osrc_hbm.at[pl.ds(w * CAP + ci * CH, CH)], srcb)
                pltpu.sync_copy(oldst_hbm.at[pl.ds(w * CAP + ci * CH, CH)], ldstb)
                if fanout == 1:
                    gidx = srcb
                else:
                    for k in range(CH // 16):
                        v = srcb[pl.ds(k * 16, 16)]
                        idxb[pl.ds(k * 16, 16)] = v * fanout + half
                    gidx = idxb
                pltpu.async_copy(h_hbm.at[gidx], rows_v, sem).wait()
                _process(acc_s, acc_m, acc_c, ldstb, rows_v, iota, one0)
                return 0

            lax.fori_loop(0, nch, chunk, 0)
        else:
            bufs = ((srcb0, ldstb0, rows0, sem0), (srcb1, ldstb1, rows1, sem1))

            def issue(ci, sb, lb, rv, sem):
                pltpu.sync_copy(osrc_hbm.at[pl.ds(w * CAP + ci * CH, CH)], sb)
                pltpu.sync_copy(oldst_hbm.at[pl.ds(w * CAP + ci * CH, CH)], lb)
                if fanout != 1:
                    for k in range(CH // 16):
                        v = sb[pl.ds(k * 16, 16)]
                        sb[pl.ds(k * 16, 16)] = v * fanout + half
                pltpu.async_copy(h_hbm.at[sb], rv, sem)

            def wait_rows(sb, rv, sem):
                pltpu.make_async_copy(h_hbm.at[sb], rv, sem).wait()

            @pl.when(nch > 0)
            def _():
                issue(0, *bufs[0])

            def pair(cp, _):
                ci0 = cp * 2
                ci1 = ci0 + 1
                ci2 = ci0 + 2

                @pl.when(ci1 < nch)
                def _():
                    issue(ci1, *bufs[1])

                wait_rows(bufs[0][0], bufs[0][2], bufs[0][3])
                _process(acc_s, acc_m, None, bufs[0][1], bufs[0][2], iota, one0)

                @pl.when(ci2 < nch)
                def _():
                    issue(ci2, *bufs[0])

                @pl.when(ci1 < nch)
                def _():
                    wait_rows(bufs[1][0], bufs[1][2], bufs[1][3])
                    _process(acc_s, acc_m, None, bufs[1][1], bufs[1][2], iota, one0)
                return 0

            lax.fori_loop(0, (nch + 1) // 2, pair, 0)
        pltpu.sync_copy(acc_s, sum_hbm.at[pl.ds(w * PR, PR)])
        pltpu.sync_copy(acc_m, max_hbm.at[pl.ds(w * PR, PR)])
        if with_cnt:
            pltpu.sync_copy(acc_c, cnt_hbm.at[pl.ds(w * PR, PR)])

    outs = [jax.ShapeDtypeStruct((NP, 128), jnp.float32),
            jax.ShapeDtypeStruct((NP, 128), jnp.float32)]
    if with_cnt:
        outs.append(jax.ShapeDtypeStruct((NP, 16), jnp.float32))
    if with_cnt:
        scratch = [
            pltpu.VMEM((CH,), jnp.int32),
            pltpu.VMEM((CH,), jnp.int32),
            pltpu.VMEM((CH,), jnp.int32),
            pltpu.VMEM((CH, 128), jnp.float32),
            pltpu.VMEM((PR, 128), jnp.float32),
            pltpu.VMEM((PR, 128), jnp.float32),
            pltpu.VMEM((PR, 16), jnp.float32),
            pltpu.VMEM((16,), jnp.int32),
            pltpu.SemaphoreType.DMA,
        ]
    else:
        scratch = [
            pltpu.VMEM((CH,), jnp.int32),
            pltpu.VMEM((CH,), jnp.int32),
            pltpu.VMEM((CH,), jnp.int32),
            pltpu.VMEM((CH,), jnp.int32),
            pltpu.VMEM((CH, 128), jnp.float32),
            pltpu.VMEM((CH, 128), jnp.float32),
            pltpu.VMEM((PR, 128), jnp.float32),
            pltpu.VMEM((PR, 128), jnp.float32),
            pltpu.VMEM((16,), jnp.int32),
            pltpu.SemaphoreType.DMA,
            pltpu.SemaphoreType.DMA,
        ]
    return pl.kernel(
        body,
        out_type=tuple(outs),
        mesh=_MESH,
        compiler_params=pltpu.CompilerParams(needs_layout_passes=False),
        scratch_types=scratch,
    )


_seg0 = _make_seg(1, 0, True)
_seg1a = _make_seg(2, 0, False)
_seg1b = _make_seg(2, 1, False)


_BNS = float(1.0 / np.sqrt(1.0 + BN_EPS))


def _sage_path(h, agg, wlT, bl, wrT, g, b):
    o = jnp.dot(agg, wlT, preferred_element_type=jnp.float32) + bl
    o = o + jnp.dot(h, wrT, preferred_element_type=jnp.float32)
    n = jnp.sqrt(jnp.sum(o * o, axis=1, keepdims=True))
    o = o / jnp.maximum(n, 1e-12)
    o = g * o * _BNS + b
    return jnp.maximum(o, 0.0)


def _tc0_body(x_ref, s_ref, m_ref, c_ref,
              wlmT, blm, wrmT, gm, bm, wlxT, blx, wrxT, gx, bx, aw,
              h_out):
    h = x_ref[...]
    c0 = c_ref[:, 0:1]
    mean = s_ref[...] / jnp.maximum(c0, 1.0)
    mx = jnp.where(c0 > 0.0, m_ref[...], 0.0)
    hm = _sage_path(h, mean, wlmT[...], blm[...], wrmT[...], gm[...], bm[...])
    hx = _sage_path(h, mx, wlxT[...], blx[...], wrxT[...], gx[...], bx[...])
    a = aw[...]
    e = jnp.exp(a - jnp.max(a))
    wgt = e / jnp.sum(e)
    h_out[...] = wgt[0, 0] * hm + wgt[0, 1] * hx


def _tc1_body(h_ref, sa_ref, sb_ref, ma_ref, mb_ref, c_ref,
              wlmT, blm, wrmT, gm, bm, wlxT, blx, wrxT, gx, bx, aw,
              cw0T, cb0, cg0, cbe0, cw1T, cb1, cg1, cbe1, cw2T, cb2,
              emb_out, log_out, acc):
    i = pl.program_id(0)

    @pl.when(i == 0)
    def _():
        acc[...] = jnp.zeros_like(acc)

    h = h_ref[...]
    c0 = c_ref[:, 0:1]
    s = jnp.concatenate([sa_ref[...], sb_ref[...]], axis=1)
    mx = jnp.concatenate([ma_ref[...], mb_ref[...]], axis=1)
    mean = s / jnp.maximum(c0, 1.0)
    mx = jnp.where(c0 > 0.0, mx, 0.0)
    hm = _sage_path(h, mean, wlmT[...], blm[...], wrmT[...], gm[...], bm[...])
    hx = _sage_path(h, mx, wlxT[...], blx[...], wrxT[...], gx[...], bx[...])
    a = aw[...]
    e = jnp.exp(a - jnp.max(a))
    wgt = e / jnp.sum(e)
    h2 = wgt[0, 0] * hm + wgt[0, 1] * hx
    acc[0:1, :] = acc[0:1, :] + jnp.sum(h2, axis=0, keepdims=True)

    @pl.when(i == NBLK - 1)
    def _():
        emb = acc[0:1, :] * (1.0 / NN)
        z = jnp.dot(emb, cw0T[...], preferred_element_type=jnp.float32) + cb0[...]
        z = jnp.maximum(cg0[...] * z * _BNS + cbe0[...], 0.0)
        z = jnp.dot(z, cw1T[...], preferred_element_type=jnp.float32) + cb1[...]
        z = jnp.maximum(cg1[...] * z * _BNS + cbe1[...], 0.0)
        log_out[...] = jnp.dot(z, cw2T[...], preferred_element_type=jnp.float32) + cb2[...]
        emb_out[...] = emb


def _full(shape):
    return pl.BlockSpec(shape, lambda i: (0, 0))


def _rows(shape):
    return pl.BlockSpec(shape, lambda i: (i, 0))


_tc0 = pl.pallas_call(
    _tc0_body,
    grid=(NBLK,),
    in_specs=[
        _rows((BR, 128)), _rows((BR, 128)), _rows((BR, 128)), _rows((BR, 16)),
        _full((128, 256)), _full((1, 256)), _full((128, 256)),
        _full((1, 256)), _full((1, 256)),
        _full((128, 256)), _full((1, 256)), _full((128, 256)),
        _full((1, 256)), _full((1, 256)),
        _full((1, 2)),
    ],
    out_specs=_rows((BR, 256)),
    out_shape=jax.ShapeDtypeStruct((NN, 256), jnp.float32),
)

_tc1 = pl.pallas_call(
    _tc1_body,
    grid=(NBLK,),
    in_specs=[
        _rows((BR, 256)), _rows((BR, 128)), _rows((BR, 128)),
        _rows((BR, 128)), _rows((BR, 128)), _rows((BR, 16)),
        _full((256, 128)), _full((1, 128)), _full((256, 128)),
        _full((1, 128)), _full((1, 128)),
        _full((256, 128)), _full((1, 128)), _full((256, 128)),
        _full((1, 128)), _full((1, 128)),
        _full((1, 2)),
        _full((128, 256)), _full((1, 256)), _full((1, 256)), _full((1, 256)),
        _full((256, 128)), _full((1, 128)), _full((1, 128)), _full((1, 128)),
        _full((128, 2)), _full((1, 2)),
    ],
    out_specs=[_full((1, 128)), _full((1, 2))],
    out_shape=[jax.ShapeDtypeStruct((1, 128), jnp.float32),
               jax.ShapeDtypeStruct((1, 2), jnp.float32)],
    scratch_shapes=[pltpu.VMEM((8, 128), jnp.float32)],
)


def _row(v):
    return v.reshape(1, -1)


def kernel(x, params, edge_index):
    src = edge_index[0].astype(jnp.int32)
    dst = edge_index[1].astype(jnp.int32)
    osrc, oldst, nch = _partition(src, dst)
    s0, m0, c0 = _seg0(x, osrc, oldst, nch)

    p = params
    h1 = _tc0(
        x, s0[:NN], m0[:NN], c0[:NN],
        p["mean0_Wl"].T, _row(p["mean0_bl"]), p["mean0_Wr"].T,
        _row(p["mean0_gamma"]), _row(p["mean0_beta"]),
        p["max0_Wl"].T, _row(p["max0_bl"]), p["max0_Wr"].T,
        _row(p["max0_gamma"]), _row(p["max0_beta"]),
        _row(p["aggr_w0"]),
    )

    h1v = h1.reshape(NN * 2, 128)
    s1a, m1a = _seg1a(h1v, osrc, oldst, nch)
    s1b, m1b = _seg1b(h1v, osrc, oldst, nch)

    emb, logits = _tc1(
        h1, s1a[:NN], s1b[:NN], m1a[:NN], m1b[:NN], c0[:NN],
        p["mean1_Wl"].T, _row(p["mean1_bl"]), p["mean1_Wr"].T,
        _row(p["mean1_gamma"]), _row(p["mean1_beta"]),
        p["max1_Wl"].T, _row(p["max1_bl"]), p["max1_Wr"].T,
        _row(p["max1_gamma"]), _row(p["max1_beta"]),
        _row(p["aggr_w1"]),
        p["cls_W0"].T, _row(p["cls_b0"]), _row(p["cls_gamma0"]), _row(p["cls_beta0"]),
        p["cls_W1"].T, _row(p["cls_b1"]), _row(p["cls_gamma1"]), _row(p["cls_beta1"]),
        p["cls_W2"].T, _row(p["cls_b2"]),
    )
    return (logits, emb)


# speedup vs baseline: 1.2601x; 1.0667x over previous
import functools

import jax
import jax.numpy as jnp
import numpy as np
from jax import lax
from jax.experimental import pallas as pl
from jax.experimental.pallas import tpu as pltpu
from jax.experimental.pallas import tpu_sc as plsc

NN = 10000
EE = 320000
DIN = 128
BN_EPS = 1e-5
NC, NS = 2, 16
NT = NC * NS
PR = 320
NP = NT * PR
CAP = 12288
CH = 48
ECH = 4000
BR = 400
NBLK = NN // BR
NEG = -3.0e38

_MESH = plsc.VectorSubcoreMesh(
    core_axis_name="c", subcore_axis_name="s", num_cores=NC, num_subcores=NS)


def _wid():
    return lax.axis_index("s") * NC + lax.axis_index("c")


def _partition_body(src_hbm, dst_hbm, osrc_hbm, oldst_hbm, nch_hbm,
                    srcb0, dstb0, srcb1, dstb1, osrc_v, oldst_v, miscv,
                    sema, semb):
    w = _wid()
    lo16 = jnp.full((16,), w * PR, jnp.int32)
    iota = lax.iota(jnp.int32, 16)
    zero16 = jnp.zeros((16,), jnp.int32)
    bufs = ((srcb0, dstb0, sema), (srcb1, dstb1, semb))

    def issue(ci, sb, db, sem):
        pltpu.async_copy(src_hbm.at[pl.ds(ci * ECH, ECH)], sb, sem)
        pltpu.async_copy(dst_hbm.at[pl.ds(ci * ECH, ECH)], db, sem)

    def wait(ci, sb, db, sem):
        pltpu.make_async_copy(src_hbm.at[pl.ds(ci * ECH, ECH)], sb, sem).wait()
        pltpu.make_async_copy(dst_hbm.at[pl.ds(ci * ECH, ECH)], db, sem).wait()

    def scan(srcb, dstb, off):
        def grp(g, off):
            d16 = dstb[pl.ds(g * 16, 16)]
            s16 = srcb[pl.ds(g * 16, 16)]
            ld = d16 - lo16
            m = (ld >= 0) & (ld < PR)
            mi = m.astype(jnp.int32)
            cs = plsc.cumsum(mi)
            pos = off + cs - mi
            ok = m & (pos < CAP - CH)
            plsc.store_scatter(osrc_v, [pos], s16, mask=ok)
            plsc.store_scatter(oldst_v, [pos], ld, mask=ok)
            return off + plsc.all_reduce_population_count(m)

        return lax.fori_loop(0, ECH // 16, grp, off)

    issue(0, *bufs[0])

    def chunk_pair(cp, off):
        ci0 = cp * 2
        ci1 = ci0 + 1
        ci2 = ci0 + 2

        @pl.when(ci1 < EE // ECH)
        def _():
            issue(ci1, *bufs[1])

        wait(ci0, *bufs[0])
        off = scan(srcb0, dstb0, off)

        @pl.when(ci2 < EE // ECH)
        def _():
            issue(ci2, *bufs[0])

        def second(off):
            wait(ci1, *bufs[1])
            return scan(srcb1, dstb1, off)

        off = lax.cond(ci1 < EE // ECH, second, lambda o: o, off)
        return off

    off = lax.fori_loop(0, (EE // ECH + 1) // 2, chunk_pair, zero16)
    pc = ((off + (CH - 1)) // CH) * CH
    for k in range(CH // 16):
        pos = off + k * 16 + iota
        pm = pos < pc
        plsc.store_scatter(osrc_v, [pos], zero16, mask=pm)
        plsc.store_scatter(oldst_v, [pos], zero16 + PR, mask=pm)
    miscv[...] = pc // CH
    pltpu.sync_copy(osrc_v, osrc_hbm.at[pl.ds(w * CAP, CAP)])
    pltpu.sync_copy(oldst_v, oldst_hbm.at[pl.ds(w * CAP, CAP)])
    pltpu.sync_copy(miscv, nch_hbm.at[pl.ds(w * 16, 16)])


_partition = pl.kernel(
    _partition_body,
    out_type=(
        jax.ShapeDtypeStruct((NT * CAP,), jnp.int32),
        jax.ShapeDtypeStruct((NT * CAP,), jnp.int32),
        jax.ShapeDtypeStruct((NT * 16,), jnp.int32),
    ),
    mesh=_MESH,
    compiler_params=pltpu.CompilerParams(needs_layout_passes=False),
    scratch_types=[
        pltpu.VMEM((ECH,), jnp.int32),
        pltpu.VMEM((ECH,), jnp.int32),
        pltpu.VMEM((ECH,), jnp.int32),
        pltpu.VMEM((ECH,), jnp.int32),
        pltpu.VMEM((CAP,), jnp.int32),
        pltpu.VMEM((CAP,), jnp.int32),
        pltpu.VMEM((16,), jnp.int32),
        pltpu.SemaphoreType.DMA,
        pltpu.SemaphoreType.DMA,
    ],
)


def _vbcast(v, j):
    idx = jnp.full((16, 1), j, jnp.int32)
    dn = lax.GatherDimensionNumbers(
        offset_dims=(), collapsed_slice_dims=(0,), start_index_map=(0,))
    return lax.gather(v, idx, dn, (1,),
                      mode=lax.GatherScatterMode.PROMISE_IN_BOUNDS)


def _make_seg(fanout, half, with_cnt):
    def _process(acc_s, acc_m, acc_c, ldstb, rows_v, iota, one0):
        def grp(g, _):
            ld16 = ldstb[pl.ds(g * 16, 16)]
            for j in range(16):
                e = g * 16 + j
                ldb = _vbcast(ld16, j)
                ok = ldb < PR
                cols = [iota + k * 16 for k in range(8)]
                mg = [plsc.load_gather(acc_m, [ldb, cols[k]], mask=ok)
                      for k in range(8)]
                rr = [rows_v[e, pl.ds(k * 16, 16)] for k in range(8)]
                for k in range(8):
                    plsc.store_scatter(acc_m, [ldb, cols[k]],
                                       jnp.maximum(mg[k], rr[k]), mask=ok)
                for k in range(8):
                    plsc.addupdate_scatter(acc_s, [ldb, cols[k]], rr[k], mask=ok)
                if acc_c is not None:
                    plsc.addupdate_scatter(acc_c, [ldb, iota], one0, mask=ok)
            return 0

        lax.fori_loop(0, CH // 16, grp, 0)

    def body(h_hbm, osrc_hbm, oldst_hbm, nch_hbm, *rest):
        if with_cnt:
            sum_hbm, max_hbm, cnt_hbm = rest[:3]
            rest = rest[3:]
        else:
            sum_hbm, max_hbm = rest[:2]
            rest = rest[2:]
        w = _wid()
        iota = lax.iota(jnp.int32, 16)
        zero16f = jnp.zeros((16,), jnp.float32)
        one0 = (iota < 1).astype(jnp.float32)

        if with_cnt:
            srcb, ldstb, idxb, rows_v = rest[:4]
            acc_s = rest[4]
            acc_m = rest[5]
            acc_c, nchv, sem = rest[6:]
        else:
            srcb0, srcb1, ldstb0, ldstb1, rows0, rows1 = rest[:6]
            acc_s, acc_m, nchv, sem0, sem1 = rest[6:]
            acc_c = None

        def init(i, _):
            for k in range(8):
                acc_s[i, pl.ds(k * 16, 16)] = zero16f
                acc_m[i, pl.ds(k * 16, 16)] = zero16f + NEG
            if acc_c is not None:
                acc_c[i, pl.ds(0, 16)] = zero16f
            return 0

        lax.fori_loop(0, PR, init, 0)
        pltpu.sync_copy(nch_hbm.at[pl.ds(w * 16, 16)], nchv)
        nch = jnp.max(nchv[...])

        if with_cnt:
            def chunk(ci, _):
                pltpu.sync_copy(osrc_hbm.at[pl.ds(w * CAP + ci * CH, CH)], srcb)
                pltpu.sync_copy(oldst_hbm.at[pl.ds(w * CAP + ci * CH, CH)], ldstb)
                if fanout == 1:
                    gidx = srcb
                else:
                    for k in range(CH // 16):
                        v = srcb[pl.ds(k * 16, 16)]
                        idxb[pl.ds(k * 16, 16)] = v * fanout + half
                    gidx = idxb
                pltpu.async_copy(h_hbm.at[gidx], rows_v, sem).wait()
                _process(acc_s, acc_m, acc_c, ldstb, rows_v, iota, one0)
                return 0

            lax.fori_loop(0, nch, chunk, 0)
        else:
            bufs = ((srcb0, ldstb0, rows0, sem0), (srcb1, ldstb1, rows1, sem1))

            def issue(ci, sb, lb, rv, sem):
                pltpu.sync_copy(osrc_hbm.at[pl.ds(w * CAP + ci * CH, CH)], sb)
                pltpu.sync_copy(oldst_hbm.at[pl.ds(w * CAP + ci * CH, CH)], lb)
                if fanout != 1:
                    for k in range(CH // 16):
                        v = sb[pl.ds(k * 16, 16)]
                        sb[pl.ds(k * 16, 16)] = v * fanout + half
                pltpu.async_copy(h_hbm.at[sb], rv, sem)

            def wait_rows(sb, rv, sem):
                pltpu.make_async_copy(h_hbm.at[sb], rv, sem).wait()

            @pl.when(nch > 0)
            def _():
                issue(0, *bufs[0])

            def pair(cp, _):
                ci0 = cp * 2
                ci1 = ci0 + 1
                ci2 = ci0 + 2

                @pl.when(ci1 < nch)
                def _():
                    issue(ci1, *bufs[1])

                wait_rows(bufs[0][0], bufs[0][2], bufs[0][3])
                _process(acc_s, acc_m, None, bufs[0][1], bufs[0][2], iota, one0)

                @pl.when(ci2 < nch)
                def _():
                    issue(ci2, *bufs[0])

                @pl.when(ci1 < nch)
                def _():
                    wait_rows(bufs[1][0], bufs[1][2], bufs[1][3])
                    _process(acc_s, acc_m, None, bufs[1][1], bufs[1][2], iota, one0)
                return 0

            lax.fori_loop(0, (nch + 1) // 2, pair, 0)
        pltpu.sync_copy(acc_s, sum_hbm.at[pl.ds(w * PR, PR)])
        pltpu.sync_copy(acc_m, max_hbm.at[pl.ds(w * PR, PR)])
        if with_cnt:
            pltpu.sync_copy(acc_c, cnt_hbm.at[pl.ds(w * PR, PR)])

    outs = [jax.ShapeDtypeStruct((NP, 128), jnp.float32),
            jax.ShapeDtypeStruct((NP, 128), jnp.float32)]
    if with_cnt:
        outs.append(jax.ShapeDtypeStruct((NP, 16), jnp.float32))
    if with_cnt:
        scratch = [
            pltpu.VMEM((CH,), jnp.int32),
            pltpu.VMEM((CH,), jnp.int32),
            pltpu.VMEM((CH,), jnp.int32),
            pltpu.VMEM((CH, 128), jnp.float32),
            pltpu.VMEM((PR, 128), jnp.float32),
            pltpu.VMEM((PR, 128), jnp.float32),
            pltpu.VMEM((PR, 16), jnp.float32),
            pltpu.VMEM((16,), jnp.int32),
            pltpu.SemaphoreType.DMA,
        ]
    else:
        scratch = [
            pltpu.VMEM((CH,), jnp.int32),
            pltpu.VMEM((CH,), jnp.int32),
            pltpu.VMEM((CH,), jnp.int32),
            pltpu.VMEM((CH,), jnp.int32),
            pltpu.VMEM((CH, 128), jnp.float32),
            pltpu.VMEM((CH, 128), jnp.float32),
            pltpu.VMEM((PR, 128), jnp.float32),
            pltpu.VMEM((PR, 128), jnp.float32),
            pltpu.VMEM((16,), jnp.int32),
            pltpu.SemaphoreType.DMA,
            pltpu.SemaphoreType.DMA,
        ]
    return pl.kernel(
        body,
        out_type=tuple(outs),
        mesh=_MESH,
        compiler_params=pltpu.CompilerParams(needs_layout_passes=False),
        scratch_types=scratch,
    )


_seg0 = _make_seg(1, 0, True)
_seg1a = _make_seg(2, 0, False)
_seg1b = _make_seg(2, 1, False)


_BNS = float(1.0 / np.sqrt(1.0 + BN_EPS))


def _sage_path(h, agg, wlT, bl, wrT, g, b):
    o = jnp.dot(agg, wlT, preferred_element_type=jnp.float32) + bl
    o = o + jnp.dot(h, wrT, preferred_element_type=jnp.float32)
    n = jnp.sqrt(jnp.sum(o * o, axis=1, keepdims=True))
    o = o / jnp.maximum(n, 1e-12)
    o = g * o * _BNS + b
    return jnp.maximum(o, 0.0)


def _tc0_body(x_ref, s_ref, m_ref, c_ref,
              wlmT, blm, wrmT, gm, bm, wlxT, blx, wrxT, gx, bx, aw,
              h_out):
    h = x_ref[...]
    c0 = c_ref[:, 0:1]
    mean = s_ref[...] / jnp.maximum(c0, 1.0)
    mx = jnp.where(c0 > 0.0, m_ref[...], 0.0)
    hm = _sage_path(h, mean, wlmT[...], blm[...], wrmT[...], gm[...], bm[...])
    hx = _sage_path(h, mx, wlxT[...], blx[...], wrxT[...], gx[...], bx[...])
    a = aw[...]
    e = jnp.exp(a - jnp.max(a))
    wgt = e / jnp.sum(e)
    h_out[...] = wgt[0, 0] * hm + wgt[0, 1] * hx


def _tc1_body(h_ref, sa_ref, sb_ref, ma_ref, mb_ref, c_ref,
              wlmT, blm, wrmT, gm, bm, wlxT, blx, wrxT, gx, bx, aw,
              cw0T, cb0, cg0, cbe0, cw1T, cb1, cg1, cbe1, cw2T, cb2,
              emb_out, log_out, acc):
    i = pl.program_id(0)

    @pl.when(i == 0)
    def _():
        acc[...] = jnp.zeros_like(acc)

    h = h_ref[...]
    c0 = c_ref[:, 0:1]
    s = jnp.concatenate([sa_ref[...], sb_ref[...]], axis=1)
    mx = jnp.concatenate([ma_ref[...], mb_ref[...]], axis=1)
    mean = s / jnp.maximum(c0, 1.0)
    mx = jnp.where(c0 > 0.0, mx, 0.0)
    hm = _sage_path(h, mean, wlmT[...], blm[...], wrmT[...], gm[...], bm[...])
    hx = _sage_path(h, mx, wlxT[...], blx[...], wrxT[...], gx[...], bx[...])
    a = aw[...]
    e = jnp.exp(a - jnp.max(a))
    wgt = e / jnp.sum(e)
    h2 = wgt[0, 0] * hm + wgt[0, 1] * hx
    acc[0:1, :] = acc[0:1, :] + jnp.sum(h2, axis=0, keepdims=True)

    @pl.when(i == NBLK - 1)
    def _():
        emb = acc[0:1, :] * (1.0 / NN)
        z = jnp.dot(emb, cw0T[...], preferred_element_type=jnp.float32) + cb0[...]
        z = jnp.maximum(cg0[...] * z * _BNS + cbe0[...], 0.0)
        z = jnp.dot(z, cw1T[...], preferred_element_type=jnp.float32) + cb1[...]
        z = jnp.maximum(cg1[...] * z * _BNS + cbe1[...], 0.0)
        log_out[...] = jnp.dot(z, cw2T[...], preferred_element_type=jnp.float32) + cb2[...]
        emb_out[...] = emb


def _full(shape):
    return pl.BlockSpec(shape, lambda i: (0, 0))


def _rows(shape):
    return pl.BlockSpec(shape, lambda i: (i, 0))


_tc0 = pl.pallas_call(
    _tc0_body,
    grid=(NBLK,),
    in_specs=[
        _rows((BR, 128)), _rows((BR, 128)), _rows((BR, 128)), _rows((BR, 16)),
        _full((128, 256)), _full((1, 256)), _full((128, 256)),
        _full((1, 256)), _full((1, 256)),
        _full((128, 256)), _full((1, 256)), _full((128, 256)),
        _full((1, 256)), _full((1, 256)),
        _full((1, 2)),
    ],
    out_specs=_rows((BR, 256)),
    out_shape=jax.ShapeDtypeStruct((NN, 256), jnp.float32),
)

_tc1 = pl.pallas_call(
    _tc1_body,
    grid=(NBLK,),
    in_specs=[
        _rows((BR, 256)), _rows((BR, 128)), _rows((BR, 128)),
        _rows((BR, 128)), _rows((BR, 128)), _rows((BR, 16)),
        _full((256, 128)), _full((1, 128)), _full((256, 128)),
        _full((1, 128)), _full((1, 128)),
        _full((256, 128)), _full((1, 128)), _full((256, 128)),
        _full((1, 128)), _full((1, 128)),
        _full((1, 2)),
        _full((128, 256)), _full((1, 256)), _full((1, 256)), _full((1, 256)),
        _full((256, 128)), _full((1, 128)), _full((1, 128)), _full((1, 128)),
        _full((128, 2)), _full((1, 2)),
    ],
    out_specs=[_full((1, 128)), _full((1, 2))],
    out_shape=[jax.ShapeDtypeStruct((1, 128), jnp.float32),
               jax.ShapeDtypeStruct((1, 2), jnp.float32)],
    scratch_shapes=[pltpu.VMEM((8, 128), jnp.float32)],
)


def _row(v):
    return v.reshape(1, -1)


def kernel(x, params, edge_index):
    src = edge_index[0].astype(jnp.int32)
    dst = edge_index[1].astype(jnp.int32)
    osrc, oldst, nch = _partition(src, dst)
    s0, m0, c0 = _seg0(x, osrc, oldst, nch)

    p = params
    h1 = _tc0(
        x, s0[:NN], m0[:NN], c0[:NN],
        p["mean0_Wl"].T, _row(p["mean0_bl"]), p["mean0_Wr"].T,
        _row(p["mean0_gamma"]), _row(p["mean0_beta"]),
        p["max0_Wl"].T, _row(p["max0_bl"]), p["max0_Wr"].T,
        _row(p["max0_gamma"]), _row(p["max0_beta"]),
        _row(p["aggr_w0"]),
    )

    h1v = h1.reshape(NN * 2, 128)
    s1a, m1a = _seg1a(h1v, osrc, oldst, nch)
    s1b, m1b = _seg1b(h1v, osrc, oldst, nch)

    emb, logits = _tc1(
        h1, s1a[:NN], s1b[:NN], m1a[:NN], m1b[:NN], c0[:NN],
        p["mean1_Wl"].T, _row(p["mean1_bl"]), p["mean1_Wr"].T,
        _row(p["mean1_gamma"]), _row(p["mean1_beta"]),
        p["max1_Wl"].T, _row(p["max1_bl"]), p["max1_Wr"].T,
        _row(p["max1_gamma"]), _row(p["max1_beta"]),
        _row(p["aggr_w1"]),
        p["cls_W0"].T, _row(p["cls_b0"]), _row(p["cls_gamma0"]), _row(p["cls_beta0"]),
        p["cls_W1"].T, _row(p["cls_b1"]), _row(p["cls_gamma1"]), _row(p["cls_beta1"]),
        p["cls_W2"].T, _row(p["cls_b2"]),
    )
    return (logits, emb)


# speedup vs baseline: 1.3914x; 1.1041x over previous
import functools

import jax
import jax.numpy as jnp
import numpy as np
from jax import lax
from jax.experimental import pallas as pl
from jax.experimental.pallas import tpu as pltpu
from jax.experimental.pallas import tpu_sc as plsc

NN = 10000
EE = 320000
DIN = 128
BN_EPS = 1e-5
NC, NS = 2, 16
NT = NC * NS
PR = 320
NP = NT * PR
CAP = 12288
CH = 48
ECH = 4000
BR = 400
NBLK = NN // BR
NEG = -3.0e38

_MESH = plsc.VectorSubcoreMesh(
    core_axis_name="c", subcore_axis_name="s", num_cores=NC, num_subcores=NS)


def _wid():
    return lax.axis_index("s") * NC + lax.axis_index("c")


def _partition_body(src_hbm, dst_hbm, osrc_hbm, oldst_hbm, nch_hbm, cnt_hbm,
                    srcb0, dstb0, srcb1, dstb1, osrc_v, oldst_v, miscv,
                    cnt1d, sema, semb):
    w = _wid()
    lo16 = jnp.full((16,), w * PR, jnp.int32)
    iota = lax.iota(jnp.int32, 16)
    zero16 = jnp.zeros((16,), jnp.int32)
    bufs = ((srcb0, dstb0, sema), (srcb1, dstb1, semb))

    def issue(ci, sb, db, sem):
        pltpu.async_copy(src_hbm.at[pl.ds(ci * ECH, ECH)], sb, sem)
        pltpu.async_copy(dst_hbm.at[pl.ds(ci * ECH, ECH)], db, sem)

    def wait(ci, sb, db, sem):
        pltpu.make_async_copy(src_hbm.at[pl.ds(ci * ECH, ECH)], sb, sem).wait()
        pltpu.make_async_copy(dst_hbm.at[pl.ds(ci * ECH, ECH)], db, sem).wait()

    def scan(srcb, dstb, off):
        def grp(g, off):
            d16 = dstb[pl.ds(g * 16, 16)]
            s16 = srcb[pl.ds(g * 16, 16)]
            ld = d16 - lo16
            m = (ld >= 0) & (ld < PR)
            mi = m.astype(jnp.int32)
            cs = plsc.cumsum(mi)
            pos = off + cs - mi
            ok = m & (pos < CAP - CH)
            plsc.store_scatter(osrc_v, [pos], s16, mask=ok)
            plsc.store_scatter(oldst_v, [pos], ld, mask=ok)
            return off + plsc.all_reduce_population_count(m)

        return lax.fori_loop(0, ECH // 16, grp, off)

    issue(0, *bufs[0])

    def chunk_pair(cp, off):
        ci0 = cp * 2
        ci1 = ci0 + 1
        ci2 = ci0 + 2

        @pl.when(ci1 < EE // ECH)
        def _():
            issue(ci1, *bufs[1])

        wait(ci0, *bufs[0])
        off = scan(srcb0, dstb0, off)

        @pl.when(ci2 < EE // ECH)
        def _():
            issue(ci2, *bufs[0])

        def second(off):
            wait(ci1, *bufs[1])
            return scan(srcb1, dstb1, off)

        off = lax.cond(ci1 < EE // ECH, second, lambda o: o, off)
        return off

    off = lax.fori_loop(0, (EE // ECH + 1) // 2, chunk_pair, zero16)
    pc = ((off + (CH - 1)) // CH) * CH
    for k in range(CH // 16):
        pos = off + k * 16 + iota
        pm = pos < pc
        plsc.store_scatter(osrc_v, [pos], zero16, mask=pm)
        plsc.store_scatter(oldst_v, [pos], zero16 + PR, mask=pm)
    miscv[...] = pc // CH

    def czero(i, _):
        cnt1d[pl.ds(i * 16, 16)] = zero16
        return 0

    lax.fori_loop(0, (PR + 16) // 16, czero, 0)
    idxm = jnp.maximum(iota - 1, 0)
    idxp = jnp.minimum(iota + 1, 15)
    nhg = jnp.max(pc) // 16

    def hist(g, _):
        ld16 = oldst_v[pl.ds(g * 16, 16)]
        sk, _sv = plsc.sort_key_val(ld16, ld16)
        prev = _vperm(sk, idxm)
        nxt = _vperm(sk, idxp)
        st = (iota == 0) | (sk != prev)
        en = ((iota == 15) | (sk != nxt)) & (sk < PR)
        rsi = plsc.cummax(jnp.where(st, iota, 0))
        rank = iota - rsi
        base = plsc.load_gather(cnt1d, [sk])
        plsc.store_scatter(cnt1d, [sk], base + rank + 1, mask=en)
        return 0

    lax.fori_loop(0, nhg, hist, 0)
    pltpu.sync_copy(cnt1d.at[pl.ds(0, PR)], cnt_hbm.at[pl.ds(w * PR, PR)])
    pltpu.sync_copy(osrc_v, osrc_hbm.at[pl.ds(w * CAP, CAP)])
    pltpu.sync_copy(oldst_v, oldst_hbm.at[pl.ds(w * CAP, CAP)])
    pltpu.sync_copy(miscv, nch_hbm.at[pl.ds(w * 16, 16)])


_partition = pl.kernel(
    _partition_body,
    out_type=(
        jax.ShapeDtypeStruct((NT * CAP,), jnp.int32),
        jax.ShapeDtypeStruct((NT * CAP,), jnp.int32),
        jax.ShapeDtypeStruct((NT * 16,), jnp.int32),
        jax.ShapeDtypeStruct((NT * PR,), jnp.int32),
    ),
    mesh=_MESH,
    compiler_params=pltpu.CompilerParams(needs_layout_passes=False),
    scratch_types=[
        pltpu.VMEM((ECH,), jnp.int32),
        pltpu.VMEM((ECH,), jnp.int32),
        pltpu.VMEM((ECH,), jnp.int32),
        pltpu.VMEM((ECH,), jnp.int32),
        pltpu.VMEM((CAP,), jnp.int32),
        pltpu.VMEM((CAP,), jnp.int32),
        pltpu.VMEM((16,), jnp.int32),
        pltpu.VMEM((PR + 16,), jnp.int32),
        pltpu.SemaphoreType.DMA,
        pltpu.SemaphoreType.DMA,
    ],
)


def _vperm(v, idx16):
    dn = lax.GatherDimensionNumbers(
        offset_dims=(), collapsed_slice_dims=(0,), start_index_map=(0,))
    return lax.gather(v, idx16.reshape(16, 1), dn, (1,),
                      mode=lax.GatherScatterMode.PROMISE_IN_BOUNDS)


def _vbcast(v, j):
    return _vperm(v, jnp.full((16,), j, jnp.int32))


def _make_seg(fanout, half, with_cnt):
    def _process(acc_s, acc_m, acc_c, ldstb, rows_v, iota, one0):
        def grp(g, _):
            ld16 = ldstb[pl.ds(g * 16, 16)]
            for j in range(16):
                e = g * 16 + j
                ldb = _vbcast(ld16, j)
                ok = ldb < PR
                cols = [iota + k * 16 for k in range(8)]
                mg = [plsc.load_gather(acc_m, [ldb, cols[k]], mask=ok)
                      for k in range(8)]
                rr = [rows_v[e, pl.ds(k * 16, 16)] for k in range(8)]
                for k in range(8):
                    plsc.store_scatter(acc_m, [ldb, cols[k]],
                                       jnp.maximum(mg[k], rr[k]), mask=ok)
                for k in range(8):
                    plsc.addupdate_scatter(acc_s, [ldb, cols[k]], rr[k], mask=ok)
                if acc_c is not None:
                    plsc.addupdate_scatter(acc_c, [ldb, iota], one0, mask=ok)
            return 0

        lax.fori_loop(0, CH // 16, grp, 0)

    def body(h_hbm, osrc_hbm, oldst_hbm, nch_hbm, *rest):
        if with_cnt:
            sum_hbm, max_hbm, cnt_hbm = rest[:3]
            rest = rest[3:]
        else:
            sum_hbm, max_hbm = rest[:2]
            rest = rest[2:]
        w = _wid()
        iota = lax.iota(jnp.int32, 16)
        zero16f = jnp.zeros((16,), jnp.float32)
        one0 = (iota < 1).astype(jnp.float32)

        if with_cnt:
            srcb, ldstb, idxb, rows_v = rest[:4]
            acc_s = rest[4]
            acc_m = rest[5]
            acc_c, nchv, sem = rest[6:]
        else:
            srcb0, srcb1, ldstb0, ldstb1, rows0, rows1 = rest[:6]
            acc_s, acc_m, nchv, sem0, sem1 = rest[6:]
            acc_c = None

        def init(i, _):
            for k in range(8):
                acc_s[i, pl.ds(k * 16, 16)] = zero16f
                acc_m[i, pl.ds(k * 16, 16)] = zero16f + NEG
            if acc_c is not None:
                acc_c[i, pl.ds(0, 16)] = zero16f
            return 0

        lax.fori_loop(0, PR, init, 0)
        pltpu.sync_copy(nch_hbm.at[pl.ds(w * 16, 16)], nchv)
        nch = jnp.max(nchv[...])

        if with_cnt:
            def chunk(ci, _):
                pltpu.sync_copy(osrc_hbm.at[pl.ds(w * CAP + ci * CH, CH)], srcb)
                pltpu.sync_copy(oldst_hbm.at[pl.ds(w * CAP + ci * CH, CH)], ldstb)
                if fanout == 1:
                    gidx = srcb
                else:
                    for k in range(CH // 16):
                        v = srcb[pl.ds(k * 16, 16)]
                        idxb[pl.ds(k * 16, 16)] = v * fanout + half
                    gidx = idxb
                pltpu.async_copy(h_hbm.at[gidx], rows_v, sem).wait()
                _process(acc_s, acc_m, acc_c, ldstb, rows_v, iota, one0)
                return 0

            lax.fori_loop(0, nch, chunk, 0)
        else:
            bufs = ((srcb0, ldstb0, rows0, sem0), (srcb1, ldstb1, rows1, sem1))

            def issue(ci, sb, lb, rv, sem):
                pltpu.sync_copy(osrc_hbm.at[pl.ds(w * CAP + ci * CH, CH)], sb)
                pltpu.sync_copy(oldst_hbm.at[pl.ds(w * CAP + ci * CH, CH)], lb)
                if fanout != 1:
                    for k in range(CH // 16):
                        v = sb[pl.ds(k * 16, 16)]
                        sb[pl.ds(k * 16, 16)] = v * fanout + half
                pltpu.async_copy(h_hbm.at[sb], rv, sem)

            def wait_rows(sb, rv, sem):
                pltpu.make_async_copy(h_hbm.at[sb], rv, sem).wait()

            @pl.when(nch > 0)
            def _():
                issue(0, *bufs[0])

            def pair(cp, _):
                ci0 = cp * 2
                ci1 = ci0 + 1
                ci2 = ci0 + 2

                @pl.when(ci1 < nch)
                def _():
                    issue(ci1, *bufs[1])

                wait_rows(bufs[0][0], bufs[0][2], bufs[0][3])
                _process(acc_s, acc_m, None, bufs[0][1], bufs[0][2], iota, one0)

                @pl.when(ci2 < nch)
                def _():
                    issue(ci2, *bufs[0])

                @pl.when(ci1 < nch)
                def _():
                    wait_rows(bufs[1][0], bufs[1][2], bufs[1][3])
                    _process(acc_s, acc_m, None, bufs[1][1], bufs[1][2], iota, one0)
                return 0

            lax.fori_loop(0, (nch + 1) // 2, pair, 0)
        pltpu.sync_copy(acc_s, sum_hbm.at[pl.ds(w * PR, PR)])
        pltpu.sync_copy(acc_m, max_hbm.at[pl.ds(w * PR, PR)])
        if with_cnt:
            pltpu.sync_copy(acc_c, cnt_hbm.at[pl.ds(w * PR, PR)])

    outs = [jax.ShapeDtypeStruct((NP, 128), jnp.float32),
            jax.ShapeDtypeStruct((NP, 128), jnp.float32)]
    if with_cnt:
        outs.append(jax.ShapeDtypeStruct((NP, 16), jnp.float32))
    if with_cnt:
        scratch = [
            pltpu.VMEM((CH,), jnp.int32),
            pltpu.VMEM((CH,), jnp.int32),
            pltpu.VMEM((CH,), jnp.int32),
            pltpu.VMEM((CH, 128), jnp.float32),
            pltpu.VMEM((PR, 128), jnp.float32),
            pltpu.VMEM((PR, 128), jnp.float32),
            pltpu.VMEM((PR, 16), jnp.float32),
            pltpu.VMEM((16,), jnp.int32),
            pltpu.SemaphoreType.DMA,
        ]
    else:
        scratch = [
            pltpu.VMEM((CH,), jnp.int32),
            pltpu.VMEM((CH,), jnp.int32),
            pltpu.VMEM((CH,), jnp.int32),
            pltpu.VMEM((CH,), jnp.int32),
            pltpu.VMEM((CH, 128), jnp.float32),
            pltpu.VMEM((CH, 128), jnp.float32),
            pltpu.VMEM((PR, 128), jnp.float32),
            pltpu.VMEM((PR, 128), jnp.float32),
            pltpu.VMEM((16,), jnp.int32),
            pltpu.SemaphoreType.DMA,
            pltpu.SemaphoreType.DMA,
        ]
    return pl.kernel(
        body,
        out_type=tuple(outs),
        mesh=_MESH,
        compiler_params=pltpu.CompilerParams(needs_layout_passes=False),
        scratch_types=scratch,
    )


_seg0 = _make_seg(1, 0, False)
_seg1a = _make_seg(2, 0, False)
_seg1b = _make_seg(2, 1, False)


_BNS = float(1.0 / np.sqrt(1.0 + BN_EPS))


def _sage_path(h, agg, wlT, bl, wrT, g, b):
    o = jnp.dot(agg, wlT, preferred_element_type=jnp.float32) + bl
    o = o + jnp.dot(h, wrT, preferred_element_type=jnp.float32)
    n = jnp.sqrt(jnp.sum(o * o, axis=1, keepdims=True))
    o = o / jnp.maximum(n, 1e-12)
    o = g * o * _BNS + b
    return jnp.maximum(o, 0.0)


def _tc0_body(x_ref, s_ref, m_ref, c_ref,
              wlmT, blm, wrmT, gm, bm, wlxT, blx, wrxT, gx, bx, aw,
              h_out):
    h = x_ref[...]
    c0 = c_ref[:, 0:1]
    mean = s_ref[...] / jnp.maximum(c0, 1.0)
    mx = jnp.where(c0 > 0.0, m_ref[...], 0.0)
    hm = _sage_path(h, mean, wlmT[...], blm[...], wrmT[...], gm[...], bm[...])
    hx = _sage_path(h, mx, wlxT[...], blx[...], wrxT[...], gx[...], bx[...])
    a = aw[...]
    e = jnp.exp(a - jnp.max(a))
    wgt = e / jnp.sum(e)
    h_out[...] = wgt[0, 0] * hm + wgt[0, 1] * hx


def _tc1_body(h_ref, sa_ref, sb_ref, ma_ref, mb_ref, c_ref,
              wlmT, blm, wrmT, gm, bm, wlxT, blx, wrxT, gx, bx, aw,
              cw0T, cb0, cg0, cbe0, cw1T, cb1, cg1, cbe1, cw2T, cb2,
              emb_out, log_out, acc):
    i = pl.program_id(0)

    @pl.when(i == 0)
    def _():
        acc[...] = jnp.zeros_like(acc)

    h = h_ref[...]
    c0 = c_ref[:, 0:1]
    s = jnp.concatenate([sa_ref[...], sb_ref[...]], axis=1)
    mx = jnp.concatenate([ma_ref[...], mb_ref[...]], axis=1)
    mean = s / jnp.maximum(c0, 1.0)
    mx = jnp.where(c0 > 0.0, mx, 0.0)
    hm = _sage_path(h, mean, wlmT[...], blm[...], wrmT[...], gm[...], bm[...])
    hx = _sage_path(h, mx, wlxT[...], blx[...], wrxT[...], gx[...], bx[...])
    a = aw[...]
    e = jnp.exp(a - jnp.max(a))
    wgt = e / jnp.sum(e)
    h2 = wgt[0, 0] * hm + wgt[0, 1] * hx
    acc[0:1, :] = acc[0:1, :] + jnp.sum(h2, axis=0, keepdims=True)

    @pl.when(i == NBLK - 1)
    def _():
        emb = acc[0:1, :] * (1.0 / NN)
        z = jnp.dot(emb, cw0T[...], preferred_element_type=jnp.float32) + cb0[...]
        z = jnp.maximum(cg0[...] * z * _BNS + cbe0[...], 0.0)
        z = jnp.dot(z, cw1T[...], preferred_element_type=jnp.float32) + cb1[...]
        z = jnp.maximum(cg1[...] * z * _BNS + cbe1[...], 0.0)
        log_out[...] = jnp.dot(z, cw2T[...], preferred_element_type=jnp.float32) + cb2[...]
        emb_out[...] = emb


def _full(shape):
    return pl.BlockSpec(shape, lambda i: (0, 0))


def _rows(shape):
    return pl.BlockSpec(shape, lambda i: (i, 0))


_tc0 = pl.pallas_call(
    _tc0_body,
    grid=(NBLK,),
    in_specs=[
        _rows((BR, 128)), _rows((BR, 128)), _rows((BR, 128)), _rows((BR, 16)),
        _full((128, 256)), _full((1, 256)), _full((128, 256)),
        _full((1, 256)), _full((1, 256)),
        _full((128, 256)), _full((1, 256)), _full((128, 256)),
        _full((1, 256)), _full((1, 256)),
        _full((1, 2)),
    ],
    out_specs=_rows((BR, 256)),
    out_shape=jax.ShapeDtypeStruct((NN, 256), jnp.float32),
)

_tc1 = pl.pallas_call(
    _tc1_body,
    grid=(NBLK,),
    in_specs=[
        _rows((BR, 256)), _rows((BR, 128)), _rows((BR, 128)),
        _rows((BR, 128)), _rows((BR, 128)), _rows((BR, 16)),
        _full((256, 128)), _full((1, 128)), _full((256, 128)),
        _full((1, 128)), _full((1, 128)),
        _full((256, 128)), _full((1, 128)), _full((256, 128)),
        _full((1, 128)), _full((1, 128)),
        _full((1, 2)),
        _full((128, 256)), _full((1, 256)), _full((1, 256)), _full((1, 256)),
        _full((256, 128)), _full((1, 128)), _full((1, 128)), _full((1, 128)),
        _full((128, 2)), _full((1, 2)),
    ],
    out_specs=[_full((1, 128)), _full((1, 2))],
    out_shape=[jax.ShapeDtypeStruct((1, 128), jnp.float32),
               jax.ShapeDtypeStruct((1, 2), jnp.float32)],
    scratch_shapes=[pltpu.VMEM((8, 128), jnp.float32)],
)


def _row(v):
    return v.reshape(1, -1)


def kernel(x, params, edge_index):
    src = edge_index[0].astype(jnp.int32)
    dst = edge_index[1].astype(jnp.int32)
    osrc, oldst, nch, cnt = _partition(src, dst)
    s0, m0 = _seg0(x, osrc, oldst, nch)
    c0 = jnp.broadcast_to(cnt[:NN].astype(jnp.float32)[:, None], (NN, 16))

    p = params
    h1 = _tc0(
        x, s0[:NN], m0[:NN], c0,
        p["mean0_Wl"].T, _row(p["mean0_bl"]), p["mean0_Wr"].T,
        _row(p["mean0_gamma"]), _row(p["mean0_beta"]),
        p["max0_Wl"].T, _row(p["max0_bl"]), p["max0_Wr"].T,
        _row(p["max0_gamma"]), _row(p["max0_beta"]),
        _row(p["aggr_w0"]),
    )

    h1v = h1.reshape(NN * 2, 128)
    s1a, m1a = _seg1a(h1v, osrc, oldst, nch)
    s1b, m1b = _seg1b(h1v, osrc, oldst, nch)

    emb, logits = _tc1(
        h1, s1a[:NN], s1b[:NN], m1a[:NN], m1b[:NN], c0,
        p["mean1_Wl"].T, _row(p["mean1_bl"]), p["mean1_Wr"].T,
        _row(p["mean1_gamma"]), _row(p["mean1_beta"]),
        p["max1_Wl"].T, _row(p["max1_bl"]), p["max1_Wr"].T,
        _row(p["max1_gamma"]), _row(p["max1_beta"]),
        _row(p["aggr_w1"]),
        p["cls_W0"].T, _row(p["cls_b0"]), _row(p["cls_gamma0"]), _row(p["cls_beta0"]),
        p["cls_W1"].T, _row(p["cls_b1"]), _row(p["cls_gamma1"]), _row(p["cls_beta1"]),
        p["cls_W2"].T, _row(p["cls_b2"]),
    )
    return (logits, emb)
